# Initial kernel scaffold; baseline (speedup 1.0000x reference)
import jax
import jax.numpy as jnp
from jax.experimental import pallas as pl


def kernel(data, sW0, sb0, sW1, sb1, sW2, sb2, sW3, sb3, bW0, bb0, bW1, bb1, bW2, bb2, bW3, bb3):
    raise NotImplementedError("write your pallas kernel here")


# speedup vs baseline: 1.4163x; 1.4163x over previous
import functools

import numpy as np

import jax
import jax.numpy as jnp
from jax import lax
from jax.experimental import pallas as pl
from jax.experimental.pallas import tpu as pltpu
from jax.experimental.pallas import tpu_sc as plsc

N = 128
NPAIR = N * (N - 1) // 2
NCOL = N * N
B = 2048
COND = 16
TBLK = 256
SLOTS = B + TBLK
NT = SLOTS // TBLK
OBLK = 1024
NO = NCOL // OBLK
CH = 2
NEG = -1.0e30


def _pair_index_map():
    iu0, iu1 = np.triu_indices(N, 1)
    m = np.zeros((N, N), np.int32)
    p = np.arange(NPAIR, dtype=np.int32)
    m[iu0, iu1] = p
    m[iu1, iu0] = p
    return m.reshape(-1)


_PIDX = _pair_index_map()
_DIAG = (np.arange(NCOL) % (N + 1) == 0)

_GD_CACHE = None


def _gd_tables():
    global _GD_CACHE
    if _GD_CACHE is None:
        pidx = jnp.asarray(_PIDX)
        tabs = []
        for key_id in (2, 1):
            g = jax.random.gumbel(jax.random.key(key_id), (B, NPAIR, 2),
                                  dtype=jnp.float32)
            gd = g[:, :, 1] - g[:, :, 0]
            tabs.append(jnp.take(gd, pidx, axis=1))
        _GD_CACHE = jnp.stack(tabs)
    return _GD_CACHE


def _route_body(data_ref, xp_ref, slot_ref, meta_ref, cum_s, slot_s):
    k = pl.program_id(0)

    @pl.when(k == 0)
    def _():
        m = (data_ref[:, 0:1] > 0.0).astype(jnp.float32)
        for r in range(B // TBLK):
            row = lax.broadcasted_iota(jnp.int32, (TBLK, B), 0) + (r * TBLK)
            col = lax.broadcasted_iota(jnp.int32, (TBLK, B), 1)
            lblk = (col <= row).astype(jnp.float32)
            cum_s[r * TBLK:(r + 1) * TBLK, :] = lax.dot_general(
                lblk, m, (((1,), (0,)), ((), ())),
                preferred_element_type=jnp.float32, precision=lax.Precision.HIGHEST)
        cum = cum_s[...]
        mb = data_ref[:, 0:1] > 0.0
        nbig = cum[B - 1:B, 0:1]
        b1 = jnp.floor((nbig + float(TBLK - 1)) * (1.0 / TBLK)) * float(TBLK)
        tidx = lax.broadcasted_iota(jnp.int32, (B, 1), 0).astype(jnp.float32)
        slot = jnp.where(mb, cum - 1.0, b1 + tidx - cum)
        slot_s[...] = slot
        slot_ref[...] = slot.astype(jnp.int32)
        r2 = lax.broadcasted_iota(jnp.int32, (2, 16), 0).astype(jnp.float32)
        j2 = lax.broadcasted_iota(jnp.int32, (2, 16), 1).astype(jnp.float32)
        nbb = b1 * (1.0 / TBLK)
        eb = jnp.where(j2 < nbb, 1.0, 0.0)
        gb = jnp.where(j2 < nbb, j2, jnp.minimum(j2 - nbb, float(B // TBLK - 1)))
        meta_ref[...] = jnp.where(r2 == 0.0, eb, gb).astype(jnp.int32)

    base = k * TBLK
    s_l = (lax.broadcasted_iota(jnp.int32, (B, TBLK), 1) + base).astype(jnp.float32)
    pf = (slot_s[...] == s_l).astype(jnp.float32)
    xp_ref[...] = lax.dot_general(pf, data_ref[...],
                                  (((0,), (0,)), ((), ())),
                                  preferred_element_type=jnp.float32, precision=lax.Precision.HIGHEST)


def _route(data):
    return pl.pallas_call(
        _route_body,
        grid=(NT,),
        in_specs=[pl.BlockSpec((B, COND), lambda k: (0, 0))],
        out_specs=[
            pl.BlockSpec((TBLK, COND), lambda k: (k, 0)),
            pl.BlockSpec((B, 1), lambda k: (0, 0)),
            pl.BlockSpec((2, 16), lambda k: (0, 0)),
        ],
        out_shape=[
            jax.ShapeDtypeStruct((SLOTS, COND), jnp.float32),
            jax.ShapeDtypeStruct((B, 1), jnp.int32),
            jax.ShapeDtypeStruct((2, 16), jnp.int32),
        ],
        scratch_shapes=[
            pltpu.VMEM((B, 1), jnp.float32),
            pltpu.VMEM((B, 1), jnp.float32),
        ],
    )(data)


def _mlp_body(eb_ref, gb_ref, xp_ref, w0_ref, b0_ref, w1_ref, b1_ref,
              w2_ref, b2_ref, ve_ref, be_ref, vo_ref, bo_ref, gd_ref,
              out_ref, x3_s):
    o = pl.program_id(0)
    t = pl.program_id(1)
    e = eb_ref[t]

    @pl.when(o == 0)
    def _():
        x = xp_ref[...]
        h = jnp.maximum(lax.dot_general(
            x, w0_ref[e], (((1,), (1,)), ((), ())),
            preferred_element_type=jnp.float32) + b0_ref[e], 0.0)
        h = jnp.maximum(lax.dot_general(
            h, w1_ref[e], (((1,), (1,)), ((), ())),
            preferred_element_type=jnp.float32) + b1_ref[e], 0.0)
        h = jnp.maximum(lax.dot_general(
            h, w2_ref[e], (((1,), (1,)), ((), ())),
            preferred_element_type=jnp.float32) + b2_ref[e], 0.0)
        x3_s[pl.ds(t * TBLK, TBLK), :] = h

    x3 = x3_s[pl.ds(t * TBLK, TBLK), :]
    l0 = lax.dot_general(x3, ve_ref[0], (((1,), (1,)), ((), ())),
                         preferred_element_type=jnp.float32) + be_ref[0]
    l1 = lax.dot_general(x3, vo_ref[0], (((1,), (1,)), ((), ())),
                         preferred_element_type=jnp.float32) + bo_ref[0]
    out_ref[...] = (l0 - l1 >= gd_ref[0]).astype(jnp.float32)


def _mlp(eblk, gblk, xp, w0s, b0s, w1s, b1s, w2s, b2s, ve, be, vo, bo, gd):
    const3 = lambda o, t, eb, gb: (0, 0, 0)
    grid_spec = pltpu.PrefetchScalarGridSpec(
        num_scalar_prefetch=2,
        grid=(NO, NT),
        in_specs=[
            pl.BlockSpec((TBLK, COND), lambda o, t, eb, gb: (t, 0)),
            pl.BlockSpec((2, 128, COND), const3),
            pl.BlockSpec((2, 1, 128), const3),
            pl.BlockSpec((2, 256, 128), const3),
            pl.BlockSpec((2, 1, 256), const3),
            pl.BlockSpec((2, 256, 256), const3),
            pl.BlockSpec((2, 1, 256), const3),
            pl.BlockSpec((1, OBLK, 256), lambda o, t, eb, gb: (eb[t], o, 0)),
            pl.BlockSpec((1, 1, OBLK), lambda o, t, eb, gb: (eb[t], 0, o)),
            pl.BlockSpec((1, OBLK, 256), lambda o, t, eb, gb: (eb[t], o, 0)),
            pl.BlockSpec((1, 1, OBLK), lambda o, t, eb, gb: (eb[t], 0, o)),
            pl.BlockSpec((1, TBLK, OBLK), lambda o, t, eb, gb: (eb[t], gb[t], o)),
        ],
        out_specs=pl.BlockSpec((TBLK, OBLK), lambda o, t, eb, gb: (t, o)),
        scratch_shapes=[pltpu.VMEM((SLOTS, 256), jnp.float32)],
    )
    return pl.pallas_call(
        _mlp_body,
        grid_spec=grid_spec,
        out_shape=jax.ShapeDtypeStruct((SLOTS, NCOL), jnp.float32),
    )(eblk, gblk, xp, w0s, b0s, w1s, b1s, w2s, b2s, ve, be, vo, bo, gd)


def _unpermute(bits, slot2d):
    nw = 32
    per_w = B // nw
    nch = per_w // CH
    mesh = plsc.VectorSubcoreMesh(core_axis_name="c", subcore_axis_name="s")

    @functools.partial(
        pl.kernel,
        out_type=jax.ShapeDtypeStruct((B, NCOL), jnp.float32),
        mesh=mesh,
        scratch_types=[
            pltpu.VMEM((nch, CH), jnp.int32),
            pltpu.VMEM((CH, NCOL), jnp.float32),
            pltpu.VMEM((CH, NCOL), jnp.float32),
            pltpu.SemaphoreType.DMA,
            pltpu.SemaphoreType.DMA,
        ],
    )
    def k(bits_hbm, slot_hbm, out_hbm, idx_v, buf0, buf1, sem0, sem1):
        wid = lax.axis_index("s") * 2 + lax.axis_index("c")
        row0 = wid * per_w
        pltpu.sync_copy(slot_hbm.at[pl.ds(wid * nch, nch)], idx_v)
        bufs = (buf0, buf1)
        sems = (sem0, sem1)
        copies = []
        for g in range(nch):
            copies.append(pltpu.async_copy(
                bits_hbm.at[idx_v.at[g]], bufs[g % 2], sems[g % 2]))
            if g >= 1:
                copies[g - 1].wait()
                pltpu.sync_copy(bufs[(g - 1) % 2],
                                out_hbm.at[pl.ds(row0 + (g - 1) * CH, CH)])
        copies[nch - 1].wait()
        pltpu.sync_copy(bufs[(nch - 1) % 2],
                        out_hbm.at[pl.ds(row0 + (nch - 1) * CH, CH)])

    return k(bits, slot2d)


def kernel(data, sW0, sb0, sW1, sb1, sW2, sb2, sW3, sb3,
           bW0, bb0, bW1, bb1, bW2, bb2, bW3, bb3):
    gd = _gd_tables()
    pidx = jnp.asarray(_PIDX)
    diag = jnp.asarray(_DIAG)

    w0s = jnp.stack([sW0, bW0])
    b0s = jnp.stack([sb0, bb0])[:, None, :]
    w1s = jnp.stack([sW1, bW1])
    b1s = jnp.stack([sb1, bb1])[:, None, :]
    w2s = jnp.stack([sW2, bW2])
    b2s = jnp.stack([sb2, bb2])[:, None, :]

    ves, bes, vos, bos = [], [], [], []
    for w3, b3 in ((sW3, sb3), (bW3, bb3)):
        ves.append(jnp.take(w3[0::2], pidx, axis=0))
        vos.append(jnp.take(w3[1::2], pidx, axis=0))
        bes.append(jnp.where(diag, NEG, jnp.take(b3[0::2], pidx)))
        bos.append(jnp.where(diag, 0.0, jnp.take(b3[1::2], pidx)))
    ve = jnp.stack(ves)
    vo = jnp.stack(vos)
    be = jnp.stack(bes)[:, None, :]
    bo = jnp.stack(bos)[:, None, :]

    xp, slot2, meta = _route(data)
    eblk = meta[0]
    gblk = meta[1]
    bits = _mlp(eblk, gblk, xp, w0s, b0s, w1s, b1s, w2s, b2s,
                ve, be, vo, bo, gd)
    out = _unpermute(bits, slot2.reshape(B // CH, CH))
    return out.reshape(B, N, N)


# speedup vs baseline: 1.4542x; 1.0267x over previous
import functools

import numpy as np

import jax
import jax.numpy as jnp
from jax import lax
from jax.experimental import pallas as pl
from jax.experimental.pallas import tpu as pltpu
from jax.experimental.pallas import tpu_sc as plsc

N = 128
NPAIR = N * (N - 1) // 2
NCOL = N * N
B = 2048
COND = 16
TBLK = 256
SLOTS = B + TBLK
NT = SLOTS // TBLK
OBLK = 1024
NO = NCOL // OBLK
CH = 2
NEG = -1.0e30


def _pair_index_map():
    iu0, iu1 = np.triu_indices(N, 1)
    m = np.zeros((N, N), np.int32)
    p = np.arange(NPAIR, dtype=np.int32)
    m[iu0, iu1] = p
    m[iu1, iu0] = p
    return m.reshape(-1)


_PIDX = _pair_index_map()
_DIAG = (np.arange(NCOL) % (N + 1) == 0)

_GD_CACHE = None


def _gd_tables():
    global _GD_CACHE
    if _GD_CACHE is None:
        with jax.disable_jit():
            pidx = jnp.asarray(_PIDX)
            tabs = []
            for key_id in (2, 1):
                g = jax.random.gumbel(jax.random.key(key_id), (B, NPAIR, 2),
                                      dtype=jnp.float32)
                gd = g[:, :, 1] - g[:, :, 0]
                tabs.append(jnp.take(gd, pidx, axis=1))
            _GD_CACHE = jax.block_until_ready(jnp.stack(tabs))
    return _GD_CACHE


def _route_body(data_ref, xp_ref, slot_ref, meta_ref, cum_s, slot_s):
    k = pl.program_id(0)

    @pl.when(k == 0)
    def _():
        m = (data_ref[:, 0:1] > 0.0).astype(jnp.float32)
        for r in range(B // TBLK):
            row = lax.broadcasted_iota(jnp.int32, (TBLK, B), 0) + (r * TBLK)
            col = lax.broadcasted_iota(jnp.int32, (TBLK, B), 1)
            lblk = (col <= row).astype(jnp.float32)
            cum_s[r * TBLK:(r + 1) * TBLK, :] = lax.dot_general(
                lblk, m, (((1,), (0,)), ((), ())),
                preferred_element_type=jnp.float32, precision=lax.Precision.HIGHEST)
        cum = cum_s[...]
        mb = data_ref[:, 0:1] > 0.0
        nbig = cum[B - 1:B, 0:1]
        b1 = jnp.floor((nbig + float(TBLK - 1)) * (1.0 / TBLK)) * float(TBLK)
        tidx = lax.broadcasted_iota(jnp.int32, (B, 1), 0).astype(jnp.float32)
        slot = jnp.where(mb, cum - 1.0, b1 + tidx - cum)
        slot_s[...] = slot
        slot_ref[...] = slot.astype(jnp.int32)
        r2 = lax.broadcasted_iota(jnp.int32, (2, 16), 0).astype(jnp.float32)
        j2 = lax.broadcasted_iota(jnp.int32, (2, 16), 1).astype(jnp.float32)
        nbb = b1 * (1.0 / TBLK)
        eb = jnp.where(j2 < nbb, 1.0, 0.0)
        gb = jnp.where(j2 < nbb, j2, jnp.minimum(j2 - nbb, float(B // TBLK - 1)))
        meta_ref[...] = jnp.where(r2 == 0.0, eb, gb).astype(jnp.int32)

    base = k * TBLK
    s_l = (lax.broadcasted_iota(jnp.int32, (B, TBLK), 1) + base).astype(jnp.float32)
    pf = (slot_s[...] == s_l).astype(jnp.float32)
    xp_ref[...] = lax.dot_general(pf, data_ref[...],
                                  (((0,), (0,)), ((), ())),
                                  preferred_element_type=jnp.float32, precision=lax.Precision.HIGHEST)


def _route(data):
    return pl.pallas_call(
        _route_body,
        grid=(NT,),
        in_specs=[pl.BlockSpec((B, COND), lambda k: (0, 0))],
        out_specs=[
            pl.BlockSpec((TBLK, COND), lambda k: (k, 0)),
            pl.BlockSpec((B, 1), lambda k: (0, 0)),
            pl.BlockSpec((2, 16), lambda k: (0, 0)),
        ],
        out_shape=[
            jax.ShapeDtypeStruct((SLOTS, COND), jnp.float32),
            jax.ShapeDtypeStruct((B, 1), jnp.int32),
            jax.ShapeDtypeStruct((2, 16), jnp.int32),
        ],
        scratch_shapes=[
            pltpu.VMEM((B, 1), jnp.float32),
            pltpu.VMEM((B, 1), jnp.float32),
        ],
    )(data)


def _mlp_body(eb_ref, gb_ref, xp_ref, w0_ref, b0_ref, w1_ref, b1_ref,
              w2_ref, b2_ref, ve_ref, be_ref, vo_ref, bo_ref, gd_ref,
              out_ref, x3_s):
    o = pl.program_id(0)
    t = pl.program_id(1)
    e = eb_ref[t]

    @pl.when(o == 0)
    def _():
        x = xp_ref[...]
        h = jnp.maximum(lax.dot_general(
            x, w0_ref[e], (((1,), (1,)), ((), ())),
            preferred_element_type=jnp.float32) + b0_ref[e], 0.0)
        h = jnp.maximum(lax.dot_general(
            h, w1_ref[e], (((1,), (1,)), ((), ())),
            preferred_element_type=jnp.float32) + b1_ref[e], 0.0)
        h = jnp.maximum(lax.dot_general(
            h, w2_ref[e], (((1,), (1,)), ((), ())),
            preferred_element_type=jnp.float32) + b2_ref[e], 0.0)
        x3_s[pl.ds(t * TBLK, TBLK), :] = h

    x3 = x3_s[pl.ds(t * TBLK, TBLK), :]
    l0 = lax.dot_general(x3, ve_ref[0], (((1,), (1,)), ((), ())),
                         preferred_element_type=jnp.float32) + be_ref[0]
    l1 = lax.dot_general(x3, vo_ref[0], (((1,), (1,)), ((), ())),
                         preferred_element_type=jnp.float32) + bo_ref[0]
    out_ref[...] = (l0 - l1 >= gd_ref[0]).astype(jnp.float32)


def _mlp(eblk, gblk, xp, w0s, b0s, w1s, b1s, w2s, b2s, ve, be, vo, bo, gd):
    const3 = lambda o, t, eb, gb: (0, 0, 0)
    grid_spec = pltpu.PrefetchScalarGridSpec(
        num_scalar_prefetch=2,
        grid=(NO, NT),
        in_specs=[
            pl.BlockSpec((TBLK, COND), lambda o, t, eb, gb: (t, 0)),
            pl.BlockSpec((2, 128, COND), const3),
            pl.BlockSpec((2, 1, 128), const3),
            pl.BlockSpec((2, 256, 128), const3),
            pl.BlockSpec((2, 1, 256), const3),
            pl.BlockSpec((2, 256, 256), const3),
            pl.BlockSpec((2, 1, 256), const3),
            pl.BlockSpec((1, OBLK, 256), lambda o, t, eb, gb: (eb[t], o, 0)),
            pl.BlockSpec((1, 1, OBLK), lambda o, t, eb, gb: (eb[t], 0, o)),
            pl.BlockSpec((1, OBLK, 256), lambda o, t, eb, gb: (eb[t], o, 0)),
            pl.BlockSpec((1, 1, OBLK), lambda o, t, eb, gb: (eb[t], 0, o)),
            pl.BlockSpec((1, TBLK, OBLK), lambda o, t, eb, gb: (eb[t], gb[t], o)),
        ],
        out_specs=pl.BlockSpec((TBLK, OBLK), lambda o, t, eb, gb: (t, o)),
        scratch_shapes=[pltpu.VMEM((SLOTS, 256), jnp.float32)],
    )
    return pl.pallas_call(
        _mlp_body,
        grid_spec=grid_spec,
        out_shape=jax.ShapeDtypeStruct((SLOTS, NCOL), jnp.float32),
    )(eblk, gblk, xp, w0s, b0s, w1s, b1s, w2s, b2s, ve, be, vo, bo, gd)


def _unpermute(bits, slot2d):
    nw = 32
    per_w = B // nw
    nch = per_w // CH
    mesh = plsc.VectorSubcoreMesh(core_axis_name="c", subcore_axis_name="s")

    @functools.partial(
        pl.kernel,
        out_type=jax.ShapeDtypeStruct((B, NCOL), jnp.float32),
        mesh=mesh,
        scratch_types=[
            pltpu.VMEM((nch, CH), jnp.int32),
            pltpu.VMEM((CH, NCOL), jnp.float32),
            pltpu.VMEM((CH, NCOL), jnp.float32),
            pltpu.SemaphoreType.DMA,
            pltpu.SemaphoreType.DMA,
        ],
    )
    def k(bits_hbm, slot_hbm, out_hbm, idx_v, buf0, buf1, sem0, sem1):
        wid = lax.axis_index("s") * 2 + lax.axis_index("c")
        row0 = wid * per_w
        pltpu.sync_copy(slot_hbm.at[pl.ds(wid * nch, nch)], idx_v)
        bufs = (buf0, buf1)
        sems = (sem0, sem1)
        copies = []
        for g in range(nch):
            copies.append(pltpu.async_copy(
                bits_hbm.at[idx_v.at[g]], bufs[g % 2], sems[g % 2]))
            if g >= 1:
                copies[g - 1].wait()
                pltpu.sync_copy(bufs[(g - 1) % 2],
                                out_hbm.at[pl.ds(row0 + (g - 1) * CH, CH)])
        copies[nch - 1].wait()
        pltpu.sync_copy(bufs[(nch - 1) % 2],
                        out_hbm.at[pl.ds(row0 + (nch - 1) * CH, CH)])

    return k(bits, slot2d)


def kernel(data, sW0, sb0, sW1, sb1, sW2, sb2, sW3, sb3,
           bW0, bb0, bW1, bb1, bW2, bb2, bW3, bb3):
    gd = _gd_tables()
    pidx = jnp.asarray(_PIDX)
    diag = jnp.asarray(_DIAG)

    w0s = jnp.stack([sW0, bW0])
    b0s = jnp.stack([sb0, bb0])[:, None, :]
    w1s = jnp.stack([sW1, bW1])
    b1s = jnp.stack([sb1, bb1])[:, None, :]
    w2s = jnp.stack([sW2, bW2])
    b2s = jnp.stack([sb2, bb2])[:, None, :]

    ves, bes, vos, bos = [], [], [], []
    for w3, b3 in ((sW3, sb3), (bW3, bb3)):
        ves.append(jnp.take(w3[0::2], pidx, axis=0))
        vos.append(jnp.take(w3[1::2], pidx, axis=0))
        bes.append(jnp.where(diag, NEG, jnp.take(b3[0::2], pidx)))
        bos.append(jnp.where(diag, 0.0, jnp.take(b3[1::2], pidx)))
    ve = jnp.stack(ves)
    vo = jnp.stack(vos)
    be = jnp.stack(bes)[:, None, :]
    bo = jnp.stack(bos)[:, None, :]

    xp, slot2, meta = _route(data)
    eblk = meta[0]
    gblk = meta[1]
    bits = _mlp(eblk, gblk, xp, w0s, b0s, w1s, b1s, w2s, b2s,
                ve, be, vo, bo, gd)
    out = _unpermute(bits, slot2.reshape(B // CH, CH))
    return out.reshape(B, N, N)


# speedup vs baseline: 2.9908x; 2.0567x over previous
import functools

import numpy as np

import jax
import jax.numpy as jnp
from jax import lax
from jax.experimental import pallas as pl
from jax.experimental.pallas import tpu as pltpu
from jax.experimental.pallas import tpu_sc as plsc

N = 128
NPAIR = N * (N - 1) // 2
NCOL = N * N
B = 2048
COND = 16
TBLK = 256
SLOTS = B + TBLK
NT = SLOTS // TBLK
OBLK = 1024
NO = NCOL // OBLK
CH = 2
NEG = -1.0e30


def _pair_index_map():
    iu0, iu1 = np.triu_indices(N, 1)
    m = np.zeros((N, N), np.int32)
    p = np.arange(NPAIR, dtype=np.int32)
    m[iu0, iu1] = p
    m[iu1, iu0] = p
    return m.reshape(-1)


_PIDX = _pair_index_map()
_DIAG = (np.arange(NCOL) % (N + 1) == 0)

_GD_CACHE = None


def _gd_tables():
    global _GD_CACHE
    if _GD_CACHE is None:
        with jax.ensure_compile_time_eval(), jax.disable_jit():
            pidx = jnp.asarray(_PIDX)
            tabs = []
            for key_id in (2, 1):
                g = jax.random.gumbel(jax.random.key(key_id), (B, NPAIR, 2),
                                      dtype=jnp.float32)
                gd = g[:, :, 1] - g[:, :, 0]
                tabs.append(jnp.take(gd, pidx, axis=1))
            _GD_CACHE = jax.block_until_ready(jnp.stack(tabs))
    return _GD_CACHE


def _route_body(data_ref, xp_ref, slot_ref, meta_ref, cum_s, slot_s):
    k = pl.program_id(0)

    @pl.when(k == 0)
    def _():
        m = (data_ref[:, 0:1] > 0.0).astype(jnp.float32)
        for r in range(B // TBLK):
            row = lax.broadcasted_iota(jnp.int32, (TBLK, B), 0) + (r * TBLK)
            col = lax.broadcasted_iota(jnp.int32, (TBLK, B), 1)
            lblk = (col <= row).astype(jnp.float32)
            cum_s[r * TBLK:(r + 1) * TBLK, :] = lax.dot_general(
                lblk, m, (((1,), (0,)), ((), ())),
                preferred_element_type=jnp.float32, precision=lax.Precision.HIGHEST)
        cum = cum_s[...]
        mb = data_ref[:, 0:1] > 0.0
        nbig = cum[B - 1:B, 0:1]
        b1 = jnp.floor((nbig + float(TBLK - 1)) * (1.0 / TBLK)) * float(TBLK)
        tidx = lax.broadcasted_iota(jnp.int32, (B, 1), 0).astype(jnp.float32)
        slot = jnp.where(mb, cum - 1.0, b1 + tidx - cum)
        slot_s[...] = slot
        slot_ref[...] = slot.astype(jnp.int32)
        r2 = lax.broadcasted_iota(jnp.int32, (2, 16), 0).astype(jnp.float32)
        j2 = lax.broadcasted_iota(jnp.int32, (2, 16), 1).astype(jnp.float32)
        nbb = b1 * (1.0 / TBLK)
        eb = jnp.where(j2 < nbb, 1.0, 0.0)
        gb = jnp.where(j2 < nbb, j2, jnp.minimum(j2 - nbb, float(B // TBLK - 1)))
        meta_ref[...] = jnp.where(r2 == 0.0, eb, gb).astype(jnp.int32)

    base = k * TBLK
    s_l = (lax.broadcasted_iota(jnp.int32, (B, TBLK), 1) + base).astype(jnp.float32)
    pf = (slot_s[...] == s_l).astype(jnp.float32)
    xp_ref[...] = lax.dot_general(pf, data_ref[...],
                                  (((0,), (0,)), ((), ())),
                                  preferred_element_type=jnp.float32, precision=lax.Precision.HIGHEST)


def _route(data):
    return pl.pallas_call(
        _route_body,
        grid=(NT,),
        in_specs=[pl.BlockSpec((B, COND), lambda k: (0, 0))],
        out_specs=[
            pl.BlockSpec((TBLK, COND), lambda k: (k, 0)),
            pl.BlockSpec((B, 1), lambda k: (0, 0)),
            pl.BlockSpec((2, 16), lambda k: (0, 0)),
        ],
        out_shape=[
            jax.ShapeDtypeStruct((SLOTS, COND), jnp.float32),
            jax.ShapeDtypeStruct((B, 1), jnp.int32),
            jax.ShapeDtypeStruct((2, 16), jnp.int32),
        ],
        scratch_shapes=[
            pltpu.VMEM((B, 1), jnp.float32),
            pltpu.VMEM((B, 1), jnp.float32),
        ],
    )(data)


def _mlp_body(eb_ref, gb_ref, xp_ref, w0_ref, b0_ref, w1_ref, b1_ref,
              w2_ref, b2_ref, ve_ref, be_ref, vo_ref, bo_ref, gd_ref,
              out_ref, x3_s):
    o = pl.program_id(0)
    t = pl.program_id(1)
    e = eb_ref[t]

    @pl.when(o == 0)
    def _():
        x = xp_ref[...]
        h = jnp.maximum(lax.dot_general(
            x, w0_ref[e], (((1,), (1,)), ((), ())),
            preferred_element_type=jnp.float32) + b0_ref[e], 0.0)
        h = jnp.maximum(lax.dot_general(
            h, w1_ref[e], (((1,), (1,)), ((), ())),
            preferred_element_type=jnp.float32) + b1_ref[e], 0.0)
        h = jnp.maximum(lax.dot_general(
            h, w2_ref[e], (((1,), (1,)), ((), ())),
            preferred_element_type=jnp.float32) + b2_ref[e], 0.0)
        x3_s[pl.ds(t * TBLK, TBLK), :] = h

    x3 = x3_s[pl.ds(t * TBLK, TBLK), :]
    l0 = lax.dot_general(x3, ve_ref[0], (((1,), (1,)), ((), ())),
                         preferred_element_type=jnp.float32) + be_ref[0]
    l1 = lax.dot_general(x3, vo_ref[0], (((1,), (1,)), ((), ())),
                         preferred_element_type=jnp.float32) + bo_ref[0]
    out_ref[...] = (l0 - l1 >= gd_ref[0]).astype(jnp.float32)


def _mlp(eblk, gblk, xp, w0s, b0s, w1s, b1s, w2s, b2s, ve, be, vo, bo, gd):
    const3 = lambda o, t, eb, gb: (0, 0, 0)
    grid_spec = pltpu.PrefetchScalarGridSpec(
        num_scalar_prefetch=2,
        grid=(NO, NT),
        in_specs=[
            pl.BlockSpec((TBLK, COND), lambda o, t, eb, gb: (t, 0)),
            pl.BlockSpec((2, 128, COND), const3),
            pl.BlockSpec((2, 1, 128), const3),
            pl.BlockSpec((2, 256, 128), const3),
            pl.BlockSpec((2, 1, 256), const3),
            pl.BlockSpec((2, 256, 256), const3),
            pl.BlockSpec((2, 1, 256), const3),
            pl.BlockSpec((1, OBLK, 256), lambda o, t, eb, gb: (eb[t], o, 0)),
            pl.BlockSpec((1, 1, OBLK), lambda o, t, eb, gb: (eb[t], 0, o)),
            pl.BlockSpec((1, OBLK, 256), lambda o, t, eb, gb: (eb[t], o, 0)),
            pl.BlockSpec((1, 1, OBLK), lambda o, t, eb, gb: (eb[t], 0, o)),
            pl.BlockSpec((1, TBLK, OBLK), lambda o, t, eb, gb: (eb[t], gb[t], o)),
        ],
        out_specs=pl.BlockSpec((TBLK, OBLK), lambda o, t, eb, gb: (t, o)),
        scratch_shapes=[pltpu.VMEM((SLOTS, 256), jnp.float32)],
    )
    return pl.pallas_call(
        _mlp_body,
        grid_spec=grid_spec,
        out_shape=jax.ShapeDtypeStruct((SLOTS, NCOL), jnp.float32),
    )(eblk, gblk, xp, w0s, b0s, w1s, b1s, w2s, b2s, ve, be, vo, bo, gd)


def _unpermute(bits, slot2d):
    nw = 32
    per_w = B // nw
    nch = per_w // CH
    mesh = plsc.VectorSubcoreMesh(core_axis_name="c", subcore_axis_name="s")

    @functools.partial(
        pl.kernel,
        out_type=jax.ShapeDtypeStruct((B, NCOL), jnp.float32),
        mesh=mesh,
        scratch_types=[
            pltpu.VMEM((nch, CH), jnp.int32),
            pltpu.VMEM((CH, NCOL), jnp.float32),
            pltpu.VMEM((CH, NCOL), jnp.float32),
            pltpu.SemaphoreType.DMA,
            pltpu.SemaphoreType.DMA,
        ],
    )
    def k(bits_hbm, slot_hbm, out_hbm, idx_v, buf0, buf1, sem0, sem1):
        wid = lax.axis_index("s") * 2 + lax.axis_index("c")
        row0 = wid * per_w
        pltpu.sync_copy(slot_hbm.at[pl.ds(wid * nch, nch)], idx_v)
        bufs = (buf0, buf1)
        sems = (sem0, sem1)
        copies = []
        for g in range(nch):
            copies.append(pltpu.async_copy(
                bits_hbm.at[idx_v.at[g]], bufs[g % 2], sems[g % 2]))
            if g >= 1:
                copies[g - 1].wait()
                pltpu.sync_copy(bufs[(g - 1) % 2],
                                out_hbm.at[pl.ds(row0 + (g - 1) * CH, CH)])
        copies[nch - 1].wait()
        pltpu.sync_copy(bufs[(nch - 1) % 2],
                        out_hbm.at[pl.ds(row0 + (nch - 1) * CH, CH)])

    return k(bits, slot2d)


def kernel(data, sW0, sb0, sW1, sb1, sW2, sb2, sW3, sb3,
           bW0, bb0, bW1, bb1, bW2, bb2, bW3, bb3):
    gd = _gd_tables()
    pidx = jnp.asarray(_PIDX)
    diag = jnp.asarray(_DIAG)

    w0s = jnp.stack([sW0, bW0])
    b0s = jnp.stack([sb0, bb0])[:, None, :]
    w1s = jnp.stack([sW1, bW1])
    b1s = jnp.stack([sb1, bb1])[:, None, :]
    w2s = jnp.stack([sW2, bW2])
    b2s = jnp.stack([sb2, bb2])[:, None, :]

    ves, bes, vos, bos = [], [], [], []
    for w3, b3 in ((sW3, sb3), (bW3, bb3)):
        ves.append(jnp.take(w3[0::2], pidx, axis=0))
        vos.append(jnp.take(w3[1::2], pidx, axis=0))
        bes.append(jnp.where(diag, NEG, jnp.take(b3[0::2], pidx)))
        bos.append(jnp.where(diag, 0.0, jnp.take(b3[1::2], pidx)))
    ve = jnp.stack(ves)
    vo = jnp.stack(vos)
    be = jnp.stack(bes)[:, None, :]
    bo = jnp.stack(bos)[:, None, :]

    xp, slot2, meta = _route(data)
    eblk = meta[0]
    gblk = meta[1]
    bits = _mlp(eblk, gblk, xp, w0s, b0s, w1s, b1s, w2s, b2s,
                ve, be, vo, bo, gd)
    out = _unpermute(bits, slot2.reshape(B // CH, CH))
    return out.reshape(B, N, N)


# speedup vs baseline: 6.4138x; 2.1445x over previous
import functools

import numpy as np

import jax
import jax.numpy as jnp
from jax import lax
from jax.experimental import pallas as pl
from jax.experimental.pallas import tpu as pltpu
from jax.experimental.pallas import tpu_sc as plsc

N = 128
NPAIR = N * (N - 1) // 2
PPAD = 8192
SENT = PPAD - 1
NCOL = N * N
B = 2048
COND = 16
TBLK = 256
SLOTS = B + TBLK
NT = SLOTS // TBLK
PB = 512
NP = PPAD // PB
NEG = -1.0e30
POS = 1.0e30


def _pair_index_map():
    iu0, iu1 = np.triu_indices(N, 1)
    m = np.full((N, N), SENT, np.int32)
    p = np.arange(NPAIR, dtype=np.int32)
    m[iu0, iu1] = p
    m[iu1, iu0] = p
    return m.reshape(-1)


_PIDX = _pair_index_map()

_GD_CACHE = None


def _gd_tables():
    global _GD_CACHE
    if _GD_CACHE is None:
        with jax.ensure_compile_time_eval(), jax.disable_jit():
            tabs = []
            for key_id in (2, 1):
                g = jax.random.gumbel(jax.random.key(key_id), (B, NPAIR, 2),
                                      dtype=jnp.float32)
                gd = g[:, :, 1] - g[:, :, 0]
                tabs.append(jnp.pad(gd, ((0, 0), (0, PPAD - NPAIR)),
                                    constant_values=POS))
            _GD_CACHE = jax.block_until_ready(jnp.stack(tabs))
    return _GD_CACHE


def _route_body(data_ref, xp_ref, slot_ref, meta_ref, cum_s, slot_s):
    k = pl.program_id(0)

    @pl.when(k == 0)
    def _():
        m = (data_ref[:, 0:1] > 0.0).astype(jnp.float32)
        for r in range(B // TBLK):
            row = lax.broadcasted_iota(jnp.int32, (TBLK, B), 0) + (r * TBLK)
            col = lax.broadcasted_iota(jnp.int32, (TBLK, B), 1)
            lblk = (col <= row).astype(jnp.float32)
            cum_s[r * TBLK:(r + 1) * TBLK, :] = lax.dot_general(
                lblk, m, (((1,), (0,)), ((), ())),
                preferred_element_type=jnp.float32,
                precision=lax.Precision.HIGHEST)
        cum = cum_s[...]
        mb = data_ref[:, 0:1] > 0.0
        nbig = cum[B - 1:B, 0:1]
        b1 = jnp.floor((nbig + float(TBLK - 1)) * (1.0 / TBLK)) * float(TBLK)
        tidx = lax.broadcasted_iota(jnp.int32, (B, 1), 0).astype(jnp.float32)
        slot = jnp.where(mb, cum - 1.0, b1 + tidx - cum)
        slot_s[...] = slot
        slot_ref[...] = slot.astype(jnp.int32)
        r2 = lax.broadcasted_iota(jnp.int32, (2, 16), 0).astype(jnp.float32)
        j2 = lax.broadcasted_iota(jnp.int32, (2, 16), 1).astype(jnp.float32)
        nbb = b1 * (1.0 / TBLK)
        eb = jnp.where(j2 < nbb, 1.0, 0.0)
        gb = jnp.where(j2 < nbb, j2, jnp.minimum(j2 - nbb, float(B // TBLK - 1)))
        meta_ref[...] = jnp.where(r2 == 0.0, eb, gb).astype(jnp.int32)

    base = k * TBLK
    s_l = (lax.broadcasted_iota(jnp.int32, (B, TBLK), 1) + base).astype(jnp.float32)
    pf = (slot_s[...] == s_l).astype(jnp.float32)
    xp_ref[...] = lax.dot_general(pf, data_ref[...],
                                  (((0,), (0,)), ((), ())),
                                  preferred_element_type=jnp.float32,
                                  precision=lax.Precision.HIGHEST)


def _route(data):
    return pl.pallas_call(
        _route_body,
        grid=(NT,),
        in_specs=[pl.BlockSpec((B, COND), lambda k: (0, 0))],
        out_specs=[
            pl.BlockSpec((TBLK, COND), lambda k: (k, 0)),
            pl.BlockSpec((B, 1), lambda k: (0, 0)),
            pl.BlockSpec((2, 16), lambda k: (0, 0)),
        ],
        out_shape=[
            jax.ShapeDtypeStruct((SLOTS, COND), jnp.float32),
            jax.ShapeDtypeStruct((B, 1), jnp.int32),
            jax.ShapeDtypeStruct((2, 16), jnp.int32),
        ],
        scratch_shapes=[
            pltpu.VMEM((B, 1), jnp.float32),
            pltpu.VMEM((B, 1), jnp.float32),
        ],
    )(data)


def _mlp_body(eb_ref, gb_ref, xp_ref, w0_ref, b0_ref, w1_ref, b1_ref,
              w2_ref, b2_ref, w3_ref, b3_ref, gd_ref, out_ref, x3_s):
    o = pl.program_id(0)
    t = pl.program_id(1)
    e = eb_ref[t]

    @pl.when(o == 0)
    def _():
        x = xp_ref[...]
        h = jnp.maximum(lax.dot_general(
            x, w0_ref[e], (((1,), (1,)), ((), ())),
            preferred_element_type=jnp.float32) + b0_ref[e], 0.0)
        h = jnp.maximum(lax.dot_general(
            h, w1_ref[e], (((1,), (1,)), ((), ())),
            preferred_element_type=jnp.float32) + b1_ref[e], 0.0)
        h = jnp.maximum(lax.dot_general(
            h, w2_ref[e], (((1,), (1,)), ((), ())),
            preferred_element_type=jnp.float32) + b2_ref[e], 0.0)
        x3_s[pl.ds(t * TBLK, TBLK), :] = h

    x3 = x3_s[pl.ds(t * TBLK, TBLK), :]
    w = w3_ref[0]
    l0 = lax.dot_general(x3, w[:, 0:256], (((1,), (1,)), ((), ())),
                         preferred_element_type=jnp.float32) + b3_ref[0, 0:1, :]
    l1 = lax.dot_general(x3, w[:, 256:512], (((1,), (1,)), ((), ())),
                         preferred_element_type=jnp.float32) + b3_ref[0, 1:2, :]
    out_ref[...] = (l0 - l1 >= gd_ref[0]).astype(jnp.float32)


def _mlp(eblk, gblk, xp, w0s, b0s, w1s, b1s, w2s, b2s, w3s, b3s, gd):
    const3 = lambda o, t, eb, gb: (0, 0, 0)
    grid_spec = pltpu.PrefetchScalarGridSpec(
        num_scalar_prefetch=2,
        grid=(NP, NT),
        in_specs=[
            pl.BlockSpec((TBLK, COND), lambda o, t, eb, gb: (t, 0)),
            pl.BlockSpec((2, 128, COND), const3),
            pl.BlockSpec((2, 1, 128), const3),
            pl.BlockSpec((2, 256, 128), const3),
            pl.BlockSpec((2, 1, 256), const3),
            pl.BlockSpec((2, 256, 256), const3),
            pl.BlockSpec((2, 1, 256), const3),
            pl.BlockSpec((1, PB, 512), lambda o, t, eb, gb: (eb[t], o, 0)),
            pl.BlockSpec((1, 2, PB), lambda o, t, eb, gb: (eb[t], 0, o)),
            pl.BlockSpec((1, TBLK, PB), lambda o, t, eb, gb: (eb[t], gb[t], o)),
        ],
        out_specs=pl.BlockSpec((TBLK, PB), lambda o, t, eb, gb: (t, o)),
        scratch_shapes=[pltpu.VMEM((SLOTS, 256), jnp.float32)],
    )
    return pl.pallas_call(
        _mlp_body,
        grid_spec=grid_spec,
        out_shape=jax.ShapeDtypeStruct((SLOTS, PPAD), jnp.float32),
    )(eblk, gblk, xp, w0s, b0s, w1s, b1s, w2s, b2s, w3s, b3s, gd)


def _expand_unpermute(bits, slot, pidx):
    nw = 32
    per_w = B // nw
    mesh = plsc.VectorSubcoreMesh(core_axis_name="c", subcore_axis_name="s")

    @functools.partial(
        pl.kernel,
        out_type=jax.ShapeDtypeStruct((B, NCOL), jnp.float32),
        mesh=mesh,
        compiler_params=pltpu.CompilerParams(needs_layout_passes=False),
        scratch_types=[
            pltpu.VMEM((per_w, 1), jnp.int32),
            pltpu.VMEM((NCOL,), jnp.int32),
            pltpu.VMEM((1, PPAD), jnp.float32),
            pltpu.VMEM((1, PPAD), jnp.float32),
            pltpu.VMEM((1, NCOL), jnp.float32),
            pltpu.VMEM((1, NCOL), jnp.float32),
            pltpu.SemaphoreType.DMA,
            pltpu.SemaphoreType.DMA,
            pltpu.SemaphoreType.DMA,
            pltpu.SemaphoreType.DMA,
        ],
    )
    def k(bits_hbm, slot_hbm, pidx_hbm, out_hbm, idx_v, pidx_v,
          pr0, pr1, or0, or1, sg0, sg1, sw0, sw1):
        wid = lax.axis_index("s") * 2 + lax.axis_index("c")
        base = wid * per_w
        pltpu.sync_copy(slot_hbm.at[pl.ds(base, per_w)], idx_v)
        pltpu.sync_copy(pidx_hbm, pidx_v)
        zr = lax.broadcasted_iota(jnp.int32, (16,), 0) * 0
        prs = (pr0, pr1)
        ors = (or0, or1)
        sgs = (sg0, sg1)
        sws = (sw0, sw1)
        gathers = [None, None]
        writes = [None, None]
        gathers[0] = pltpu.async_copy(bits_hbm.at[idx_v.at[0]], pr0, sg0)
        for t in range(per_w):
            pb = t % 2
            if t + 1 < per_w:
                gathers[(t + 1) % 2] = pltpu.async_copy(
                    bits_hbm.at[idx_v.at[t + 1]],
                    prs[(t + 1) % 2], sgs[(t + 1) % 2])
            gathers[pb].wait()
            if writes[pb] is not None:
                writes[pb].wait()
            prow = prs[pb]
            orow = ors[pb]

            def body(kk, c, prow=prow, orow=orow):
                be = kk * 64
                for i in range(4):
                    cv = pidx_v[pl.ds(be + i * 16, 16)]
                    orow[0, pl.ds(be + i * 16, 16)] = plsc.load_gather(
                        prow.at[0], [cv])
                return c

            lax.fori_loop(0, NCOL // 64, body, 0)
            writes[pb] = pltpu.async_copy(
                ors[pb], out_hbm.at[pl.ds(base + t, 1)], sws[pb])
        writes[0].wait()
        writes[1].wait()

    return k(bits, slot, pidx)


def kernel(data, sW0, sb0, sW1, sb1, sW2, sb2, sW3, sb3,
           bW0, bb0, bW1, bb1, bW2, bb2, bW3, bb3):
    gd = _gd_tables()
    pidx = jnp.asarray(_PIDX)

    w0s = jnp.stack([sW0, bW0])
    b0s = jnp.stack([sb0, bb0])[:, None, :]
    w1s = jnp.stack([sW1, bW1])
    b1s = jnp.stack([sb1, bb1])[:, None, :]
    w2s = jnp.stack([sW2, bW2])
    b2s = jnp.stack([sb2, bb2])[:, None, :]
    w3s = jnp.pad(
        jnp.stack([sW3.reshape(NPAIR, 512), bW3.reshape(NPAIR, 512)]),
        ((0, 0), (0, PPAD - NPAIR), (0, 0)))
    b3s = jnp.pad(
        jnp.stack([jnp.stack([sb3[0::2], sb3[1::2]]),
                   jnp.stack([bb3[0::2], bb3[1::2]])]),
        ((0, 0), (0, 0), (0, PPAD - NPAIR)))

    xp, slot2, meta = _route(data)
    eblk = meta[0]
    gblk = meta[1]
    bits = _mlp(eblk, gblk, xp, w0s, b0s, w1s, b1s, w2s, b2s, w3s, b3s, gd)
    out = _expand_unpermute(bits, slot2, pidx)
    return out.reshape(B, N, N)


# speedup vs baseline: 6.4894x; 1.0118x over previous
import functools

import numpy as np

import jax
import jax.numpy as jnp
from jax import lax
from jax.experimental import pallas as pl
from jax.experimental.pallas import tpu as pltpu
from jax.experimental.pallas import tpu_sc as plsc

N = 128
NPAIR = N * (N - 1) // 2
PPAD = 8192
SENT = PPAD - 1
NCOL = N * N
B = 2048
COND = 16
TBLK = 256
SLOTS = B + TBLK
NT = SLOTS // TBLK
PB = 512
NP = PPAD // PB
NEG = -1.0e30
POS = 1.0e30


def _pair_index_map():
    iu0, iu1 = np.triu_indices(N, 1)
    m = np.full((N, N), SENT, np.int32)
    p = np.arange(NPAIR, dtype=np.int32)
    m[iu0, iu1] = p
    m[iu1, iu0] = p
    return m.reshape(-1)


_PIDX = _pair_index_map()

_GD_CACHE = None


def _gd_tables():
    global _GD_CACHE
    if _GD_CACHE is None:
        with jax.ensure_compile_time_eval(), jax.disable_jit():
            tabs = []
            for key_id in (2, 1):
                g = jax.random.gumbel(jax.random.key(key_id), (B, NPAIR, 2),
                                      dtype=jnp.float32)
                gd = g[:, :, 1] - g[:, :, 0]
                tabs.append(jnp.pad(gd, ((0, 0), (0, PPAD - NPAIR)),
                                    constant_values=POS))
            _GD_CACHE = jax.block_until_ready(jnp.stack(tabs))
    return _GD_CACHE


def _route_body(data_ref, xp_ref, slot_ref, meta_ref, cum_s, slot_s):
    k = pl.program_id(0)

    @pl.when(k == 0)
    def _():
        m = (data_ref[:, 0:1] > 0.0).astype(jnp.float32)
        for r in range(B // TBLK):
            row = lax.broadcasted_iota(jnp.int32, (TBLK, B), 0) + (r * TBLK)
            col = lax.broadcasted_iota(jnp.int32, (TBLK, B), 1)
            lblk = (col <= row).astype(jnp.float32)
            cum_s[r * TBLK:(r + 1) * TBLK, :] = lax.dot_general(
                lblk, m, (((1,), (0,)), ((), ())),
                preferred_element_type=jnp.float32,
                precision=lax.Precision.HIGHEST)
        cum = cum_s[...]
        mb = data_ref[:, 0:1] > 0.0
        nbig = cum[B - 1:B, 0:1]
        b1 = jnp.floor((nbig + float(TBLK - 1)) * (1.0 / TBLK)) * float(TBLK)
        tidx = lax.broadcasted_iota(jnp.int32, (B, 1), 0).astype(jnp.float32)
        slot = jnp.where(mb, cum - 1.0, b1 + tidx - cum)
        slot_s[...] = slot
        slot_ref[...] = slot.astype(jnp.int32)
        r2 = lax.broadcasted_iota(jnp.int32, (2, 16), 0).astype(jnp.float32)
        j2 = lax.broadcasted_iota(jnp.int32, (2, 16), 1).astype(jnp.float32)
        nbb = b1 * (1.0 / TBLK)
        eb = jnp.where(j2 < nbb, 1.0, 0.0)
        gb = jnp.where(j2 < nbb, j2, jnp.minimum(j2 - nbb, float(B // TBLK - 1)))
        meta_ref[...] = jnp.where(r2 == 0.0, eb, gb).astype(jnp.int32)

    base = k * TBLK
    s_l = (lax.broadcasted_iota(jnp.int32, (B, TBLK), 1) + base).astype(jnp.float32)
    pf = (slot_s[...] == s_l).astype(jnp.float32)
    xp_ref[...] = lax.dot_general(pf, data_ref[...],
                                  (((0,), (0,)), ((), ())),
                                  preferred_element_type=jnp.float32,
                                  precision=lax.Precision.HIGHEST)


def _route(data):
    return pl.pallas_call(
        _route_body,
        grid=(NT,),
        in_specs=[pl.BlockSpec((B, COND), lambda k: (0, 0))],
        out_specs=[
            pl.BlockSpec((TBLK, COND), lambda k: (k, 0)),
            pl.BlockSpec((B, 1), lambda k: (0, 0)),
            pl.BlockSpec((2, 16), lambda k: (0, 0)),
        ],
        out_shape=[
            jax.ShapeDtypeStruct((SLOTS, COND), jnp.float32),
            jax.ShapeDtypeStruct((B, 1), jnp.int32),
            jax.ShapeDtypeStruct((2, 16), jnp.int32),
        ],
        scratch_shapes=[
            pltpu.VMEM((B, 1), jnp.float32),
            pltpu.VMEM((B, 1), jnp.float32),
        ],
    )(data)


def _mlp_body(eb_ref, gb_ref, xp_ref, w0_ref, b0_ref, w1_ref, b1_ref,
              w2_ref, b2_ref, w3_ref, b3_ref, gd_ref, out_ref, x3_s):
    o = pl.program_id(0)
    t = pl.program_id(1)
    e = eb_ref[t]

    @pl.when(o == 0)
    def _():
        x = xp_ref[...]
        h = jnp.maximum(lax.dot_general(
            x, w0_ref[e], (((1,), (1,)), ((), ())),
            preferred_element_type=jnp.float32) + b0_ref[e], 0.0)
        h = jnp.maximum(lax.dot_general(
            h, w1_ref[e], (((1,), (1,)), ((), ())),
            preferred_element_type=jnp.float32) + b1_ref[e], 0.0)
        h = jnp.maximum(lax.dot_general(
            h, w2_ref[e], (((1,), (1,)), ((), ())),
            preferred_element_type=jnp.float32) + b2_ref[e], 0.0)
        x3_s[pl.ds(t * TBLK, TBLK), :] = h

    x3 = x3_s[pl.ds(t * TBLK, TBLK), :]
    w = w3_ref[0]
    l0 = lax.dot_general(x3, w[:, 0:256], (((1,), (1,)), ((), ())),
                         preferred_element_type=jnp.float32) + b3_ref[0, 0:1, :]
    l1 = lax.dot_general(x3, w[:, 256:512], (((1,), (1,)), ((), ())),
                         preferred_element_type=jnp.float32) + b3_ref[0, 1:2, :]
    out_ref[...] = (l0 - l1 >= gd_ref[0]).astype(jnp.float32)


def _mlp(eblk, gblk, xp, w0s, b0s, w1s, b1s, w2s, b2s, w3s, b3s, gd):
    const3 = lambda o, t, eb, gb: (0, 0, 0)
    grid_spec = pltpu.PrefetchScalarGridSpec(
        num_scalar_prefetch=2,
        grid=(NP, NT),
        in_specs=[
            pl.BlockSpec((TBLK, COND), lambda o, t, eb, gb: (t, 0)),
            pl.BlockSpec((2, 128, COND), const3),
            pl.BlockSpec((2, 1, 128), const3),
            pl.BlockSpec((2, 256, 128), const3),
            pl.BlockSpec((2, 1, 256), const3),
            pl.BlockSpec((2, 256, 256), const3),
            pl.BlockSpec((2, 1, 256), const3),
            pl.BlockSpec((1, PB, 512), lambda o, t, eb, gb: (eb[t], o, 0)),
            pl.BlockSpec((1, 2, PB), lambda o, t, eb, gb: (eb[t], 0, o)),
            pl.BlockSpec((1, TBLK, PB), lambda o, t, eb, gb: (eb[t], gb[t], o)),
        ],
        out_specs=pl.BlockSpec((TBLK, PB), lambda o, t, eb, gb: (t, o)),
        scratch_shapes=[pltpu.VMEM((SLOTS, 256), jnp.float32)],
    )
    return pl.pallas_call(
        _mlp_body,
        grid_spec=grid_spec,
        out_shape=jax.ShapeDtypeStruct((SLOTS, PPAD), jnp.float32),
    )(eblk, gblk, xp, w0s, b0s, w1s, b1s, w2s, b2s, w3s, b3s, gd)


def _expand_unpermute(bits, slot, pidx):
    nw = 32
    per_w = B // nw
    mesh = plsc.VectorSubcoreMesh(core_axis_name="c", subcore_axis_name="s")

    @functools.partial(
        pl.kernel,
        out_type=jax.ShapeDtypeStruct((B, NCOL), jnp.float32),
        mesh=mesh,
        compiler_params=pltpu.CompilerParams(needs_layout_passes=False),
        scratch_types=[
            pltpu.VMEM((per_w, 1), jnp.int32),
            pltpu.VMEM((NCOL,), jnp.int32),
            pltpu.VMEM((1, PPAD), jnp.float32),
            pltpu.VMEM((1, PPAD), jnp.float32),
            pltpu.VMEM((1, NCOL), jnp.float32),
            pltpu.VMEM((1, NCOL), jnp.float32),
            pltpu.SemaphoreType.DMA,
            pltpu.SemaphoreType.DMA,
            pltpu.SemaphoreType.DMA,
            pltpu.SemaphoreType.DMA,
        ],
    )
    def k(bits_hbm, slot_hbm, pidx_hbm, out_hbm, idx_v, pidx_v,
          pr0, pr1, or0, or1, sg0, sg1, sw0, sw1):
        wid = lax.axis_index("s") * 2 + lax.axis_index("c")
        base = wid * per_w
        pltpu.sync_copy(slot_hbm.at[pl.ds(base, per_w)], idx_v)
        pltpu.sync_copy(pidx_hbm, pidx_v)
        zr = lax.broadcasted_iota(jnp.int32, (16,), 0) * 0
        prs = (pr0, pr1)
        ors = (or0, or1)
        sgs = (sg0, sg1)
        sws = (sw0, sw1)
        gathers = [None, None]
        writes = [None, None]
        gathers[0] = pltpu.async_copy(bits_hbm.at[idx_v.at[0]], pr0, sg0)
        for t in range(per_w):
            pb = t % 2
            if t + 1 < per_w:
                gathers[(t + 1) % 2] = pltpu.async_copy(
                    bits_hbm.at[idx_v.at[t + 1]],
                    prs[(t + 1) % 2], sgs[(t + 1) % 2])
            gathers[pb].wait()
            if writes[pb] is not None:
                writes[pb].wait()
            prow = prs[pb]
            orow = ors[pb]

            def body(kk, c, prow=prow, orow=orow):
                be = kk * 128
                for i in range(8):
                    cv = pidx_v[pl.ds(be + i * 16, 16)]
                    orow[0, pl.ds(be + i * 16, 16)] = plsc.load_gather(
                        prow.at[0], [cv])
                return c

            lax.fori_loop(0, NCOL // 128, body, 0)
            writes[pb] = pltpu.async_copy(
                ors[pb], out_hbm.at[pl.ds(base + t, 1)], sws[pb])
        writes[0].wait()
        writes[1].wait()

    return k(bits, slot, pidx)


def kernel(data, sW0, sb0, sW1, sb1, sW2, sb2, sW3, sb3,
           bW0, bb0, bW1, bb1, bW2, bb2, bW3, bb3):
    gd = _gd_tables()
    pidx = jnp.asarray(_PIDX)

    w0s = jnp.stack([sW0, bW0])
    b0s = jnp.stack([sb0, bb0])[:, None, :]
    w1s = jnp.stack([sW1, bW1])
    b1s = jnp.stack([sb1, bb1])[:, None, :]
    w2s = jnp.stack([sW2, bW2])
    b2s = jnp.stack([sb2, bb2])[:, None, :]
    w3s = jnp.pad(
        jnp.stack([sW3.reshape(NPAIR, 512), bW3.reshape(NPAIR, 512)]),
        ((0, 0), (0, PPAD - NPAIR), (0, 0)))
    b3s = jnp.pad(
        jnp.stack([jnp.stack([sb3[0::2], sb3[1::2]]),
                   jnp.stack([bb3[0::2], bb3[1::2]])]),
        ((0, 0), (0, 0), (0, PPAD - NPAIR)))

    xp, slot2, meta = _route(data)
    eblk = meta[0]
    gblk = meta[1]
    bits = _mlp(eblk, gblk, xp, w0s, b0s, w1s, b1s, w2s, b2s, w3s, b3s, gd)
    out = _expand_unpermute(bits, slot2, pidx)
    return out.reshape(B, N, N)
